# baseline (device time: 37059 ns/iter reference)
import jax
import jax.numpy as jnp
from jax import lax
from jax.experimental import pallas as pl
from jax.experimental.pallas import tpu as pltpu

N_DEV = 4
B, SQ, D = 2, 128, 512
H_LOC, DH = 8, 64


def kernel(x, Wq, Wo, Wk, Wv):
    def body(x_ref, wq_ref, wo_ref, wk_ref, wv_ref, out_ref,
             comm_ref, send_sems, recv_sems):
        my = lax.axis_index("i")
        left = (my + N_DEV - 1) % N_DEV
        right = (my + 1) % N_DEV

        barrier_sem = pltpu.get_barrier_semaphore()
        for nbr in (left, right):
            pl.semaphore_signal(
                barrier_sem, inc=1,
                device_id=(nbr,), device_id_type=pl.DeviceIdType.MESH,
            )
        pl.semaphore_wait(barrier_sem, 2)

        wq = wq_ref[:, :]
        wk = wk_ref[:, :]
        wv = wv_ref[:, :]
        wo = wo_ref[:, :]
        for b in range(B):
            xb = x_ref[b]
            q = jnp.dot(xb, wq, preferred_element_type=jnp.float32)
            k = jnp.dot(xb, wk, preferred_element_type=jnp.float32)
            v = jnp.dot(xb, wv, preferred_element_type=jnp.float32)
            outs = []
            for h in range(H_LOC):
                sl = slice(h * DH, (h + 1) * DH)
                qh = q[:, sl]
                kh = k[:, sl]
                vh = v[:, sl]
                s = jnp.dot(qh, kh.T, preferred_element_type=jnp.float32) * 0.125
                m = jnp.max(s, axis=-1, keepdims=True)
                p = jnp.exp(s - m)
                l = jnp.sum(p, axis=-1, keepdims=True)
                outs.append(jnp.dot(p, vh, preferred_element_type=jnp.float32) / l)
            attn = jnp.concatenate(outs, axis=1)
            partial = jnp.dot(attn, wo, preferred_element_type=jnp.float32)
            comm_ref[0, b] = partial
            out_ref[b] = partial

        for hop in range(N_DEV - 1):
            rdma = pltpu.make_async_remote_copy(
                src_ref=comm_ref.at[hop],
                dst_ref=comm_ref.at[hop + 1],
                send_sem=send_sems.at[hop],
                recv_sem=recv_sems.at[hop],
                device_id=(right,),
                device_id_type=pl.DeviceIdType.MESH,
            )
            rdma.start()
            rdma.wait()
            for b in range(B):
                out_ref[b] = out_ref[b] + comm_ref[hop + 1, b]

    return pl.pallas_call(
        body,
        out_shape=jax.ShapeDtypeStruct((B, SQ, D), jnp.float32),
        in_specs=[pl.BlockSpec(memory_space=pltpu.VMEM)] * 5,
        out_specs=pl.BlockSpec(memory_space=pltpu.VMEM),
        scratch_shapes=[
            pltpu.VMEM((N_DEV, B, SQ, D), jnp.float32),
            pltpu.SemaphoreType.DMA((N_DEV - 1,)),
            pltpu.SemaphoreType.DMA((N_DEV - 1,)),
        ],
        compiler_params=pltpu.CompilerParams(collective_id=0),
    )(x, Wq, Wo, Wk, Wv)
